# Initial kernel scaffold; baseline (speedup 1.0000x reference)
import jax
import jax.numpy as jnp
from jax.experimental import pallas as pl


def kernel(x, edge_index, batch, W1l, b1l, W1r, W2l, b2l, W2r, W3l, b3l, W3r, p1, p2, p3, lin1_W, lin1_b, lin2_W, lin2_b, lin3_W, lin3_b):
    raise NotImplementedError("write your pallas kernel here")


# speedup vs baseline: 1.0891x; 1.0891x over previous
import jax
import jax.numpy as jnp
from jax.experimental import pallas as pl

_NG = 64
_RATIO = 0.5
_NEG = -jnp.inf


def _sage_block(msum_ref, cnt_ref, x_ref, Wl_ref, bl_ref, Wr_ref, pn_ref,
                h_ref, s_ref):
    cnt = cnt_ref[...]
    agg = msum_ref[...] / jnp.maximum(cnt, 1.0)
    lhs = jax.lax.dot_general(agg, Wl_ref[...], (((1,), (1,)), ((), ())),
                              preferred_element_type=jnp.float32)
    rhs = jax.lax.dot_general(x_ref[...], Wr_ref[...], (((1,), (1,)), ((), ())),
                              preferred_element_type=jnp.float32)
    h = jnp.maximum(lhs + bl_ref[...] + rhs, 0.0)
    h_ref[...] = h
    s_ref[...] = jnp.tanh(
        jax.lax.dot_general(h, pn_ref[...], (((1,), (0,)), ((), ())),
                            preferred_element_type=jnp.float32))


def _sage_score(msum, cnt, x, Wl, bl, Wr, p):
    n, d = x.shape
    blk = 1000
    pn = (p / jnp.linalg.norm(p)).reshape(d, 1)
    h, s = pl.pallas_call(
        _sage_block,
        grid=(n // blk,),
        in_specs=[
            pl.BlockSpec((blk, d), lambda i: (i, 0)),
            pl.BlockSpec((blk, 1), lambda i: (i, 0)),
            pl.BlockSpec((blk, d), lambda i: (i, 0)),
            pl.BlockSpec((d, d), lambda i: (0, 0)),
            pl.BlockSpec((1, d), lambda i: (0, 0)),
            pl.BlockSpec((d, d), lambda i: (0, 0)),
            pl.BlockSpec((d, 1), lambda i: (0, 0)),
        ],
        out_specs=[
            pl.BlockSpec((blk, d), lambda i: (i, 0)),
            pl.BlockSpec((blk, 1), lambda i: (i, 0)),
        ],
        out_shape=[
            jax.ShapeDtypeStruct((n, d), jnp.float32),
            jax.ShapeDtypeStruct((n, 1), jnp.float32),
        ],
    )(msum, cnt.reshape(n, 1), x, Wl, bl.reshape(1, d), Wr, pn)
    return h, s[:, 0]


def _pool_block(h_ref, b_ref, out_ref):
    h = h_ref[...]
    b = b_ref[...]
    d = h.shape[1]
    rows = jax.lax.broadcasted_iota(jnp.int32, (_NG, d), 0)

    def body(g, carry):
        mx, sm, ct = carry
        mask = b == g
        sel = rows == g
        m = jnp.max(jnp.where(mask, h, _NEG), axis=0, keepdims=True)
        s = jnp.sum(jnp.where(mask, h, 0.0), axis=0, keepdims=True)
        c = jnp.sum(mask.astype(jnp.float32), axis=0, keepdims=True)
        return (jnp.where(sel, m, mx), jnp.where(sel, s, sm),
                jnp.where(sel, c, ct))

    init = (jnp.full((_NG, d), _NEG, jnp.float32),
            jnp.zeros((_NG, d), jnp.float32), jnp.zeros((_NG, d), jnp.float32))
    mx, sm, ct = jax.lax.fori_loop(0, _NG, body, init)
    out_ref[:, 0:d] = mx
    out_ref[:, d:] = sm / jnp.maximum(ct, 1.0)


def _gmp_gap(h, b):
    n, d = h.shape
    return pl.pallas_call(
        _pool_block,
        out_shape=jax.ShapeDtypeStruct((_NG, 2 * d), jnp.float32),
    )(h, b.reshape(n, 1))


def _mlp_block(z_ref, W1_ref, b1_ref, W2_ref, b2_ref, W3_ref, b3_ref, o_ref):
    z = jax.lax.dot_general(z_ref[...], W1_ref[...], (((1,), (1,)), ((), ())),
                            preferred_element_type=jnp.float32)
    z = jnp.maximum(z + b1_ref[...], 0.0)
    z = jax.lax.dot_general(z, W2_ref[...], (((1,), (1,)), ((), ())),
                            preferred_element_type=jnp.float32)
    z = jnp.maximum(z + b2_ref[...], 0.0)
    logits = jax.lax.dot_general(z, W3_ref[...], (((1,), (1,)), ((), ())),
                                 preferred_element_type=jnp.float32)
    logits = logits + b3_ref[...]
    m = jnp.max(logits, axis=1, keepdims=True)
    e = jnp.exp(logits - m)
    lse = jnp.log(jnp.sum(e, axis=1, keepdims=True))
    o_ref[...] = logits - m - lse


def _classifier(z, W1, b1, W2, b2, W3, b3):
    nc = W3.shape[0]
    return pl.pallas_call(
        _mlp_block,
        out_shape=jax.ShapeDtypeStruct((_NG, nc), jnp.float32),
    )(z, W1, b1.reshape(1, -1), W2, b2.reshape(1, -1), W3, b3.reshape(1, -1))


def _edge_agg(h, ei):
    n = h.shape[0]
    src, dst = ei[0], ei[1]
    msum = jax.ops.segment_sum(h[src], dst, num_segments=n)
    cnt = jax.ops.segment_sum(jnp.ones(ei.shape[1], jnp.float32), dst,
                              num_segments=n)
    return msum, cnt


def _topk_pool(h, score, ei, batch, valid):
    n = h.shape[0]
    vb = jnp.where(valid, batch, _NG)
    counts = jax.ops.segment_sum(valid.astype(jnp.int32), vb, num_segments=_NG)
    k = jnp.ceil(_RATIO * counts.astype(jnp.float32)).astype(jnp.int32)
    sort_key = jnp.where(valid, batch.astype(jnp.float32) * 4.0 - score,
                         4.0 * _NG + 1.0)
    order = jnp.argsort(sort_key)
    starts = jnp.concatenate(
        [jnp.zeros((1,), counts.dtype), jnp.cumsum(counts)[:-1]])
    ob = jnp.clip(vb[order], 0, _NG - 1)
    pos = jnp.arange(n) - starts[ob]
    keepc = (pos < k[ob]) & valid[order]
    keepmask = jnp.zeros((n,), bool).at[order].set(keepc)
    new_idx = jnp.cumsum(keepmask.astype(jnp.int32)) - 1
    m2 = jnp.sum(keepmask.astype(jnp.int32))
    tgt = jnp.where(keepmask, new_idx, n)
    h2 = jnp.zeros_like(h).at[tgt].set(h * score[:, None], mode='drop')
    batch2 = jnp.full((n,), _NG, batch.dtype).at[tgt].set(batch, mode='drop')
    e0 = jnp.clip(ei[0], 0, n - 1)
    e1 = jnp.clip(ei[1], 0, n - 1)
    em = keepmask[e0] & keepmask[e1] & (ei[0] < n) & (ei[1] < n)
    ei2 = jnp.stack([jnp.where(em, new_idx[e0], n),
                     jnp.where(em, new_idx[e1], n)])
    valid2 = jnp.arange(n) < m2
    return h2, ei2, batch2, valid2


def kernel(x, edge_index, batch, W1l, b1l, W1r, W2l, b2l, W2r, W3l, b3l, W3r,
           p1, p2, p3, lin1_W, lin1_b, lin2_W, lin2_b, lin3_W, lin3_b):
    n = x.shape[0]
    valid = jnp.ones((n,), bool)

    msum, cnt = _edge_agg(x, edge_index)
    h, s = _sage_score(msum, cnt, x, W1l, b1l, W1r, p1)
    h, ei, b, valid = _topk_pool(h, s, edge_index, batch, valid)
    x1 = _gmp_gap(h, b)

    msum, cnt = _edge_agg(h, ei)
    h2, s = _sage_score(msum, cnt, h, W2l, b2l, W2r, p2)
    h, ei, b, valid = _topk_pool(h2, s, ei, b, valid)
    x2 = _gmp_gap(h, b)

    msum, cnt = _edge_agg(h, ei)
    h3, s = _sage_score(msum, cnt, h, W3l, b3l, W3r, p3)
    h, ei, b, valid = _topk_pool(h3, s, ei, b, valid)
    x3 = _gmp_gap(h, b)

    z = x1 + x2 + x3
    return _classifier(z, lin1_W, lin1_b, lin2_W, lin2_b, lin3_W, lin3_b)
